# baseline (speedup 1.0000x reference)
import functools

import jax
import jax.numpy as jnp
from jax import lax
from jax.experimental import pallas as pl
from jax.experimental.pallas import tpu as pltpu
from jax.experimental.pallas import tpu_sc as plsc

N = 10000
E = 320000
D_IN = 128
HID = 64
H1 = 8
C2 = 2
W1AUG = 80
W2AUG = 16

NS = 16
NC = 2
NP = 10240
BN = 1024
NB = NP // BN

K = 128
EP = 327680
E1T = EP // NS
E2T = EP // (NC * NS)
NCH1 = E1T // K
NCH2 = E2T // K
RPT = NP // NS


def _ka_body(x_ref, w1_ref, as_ref, ad_ref, hbf_ref, asrc_ref, adst_ref):
    i = pl.program_id(0)
    h = jnp.dot(x_ref[...], w1_ref[...], preferred_element_type=jnp.float32)
    for hh in range(H1):
        hcol = h[:, hh * HID:(hh + 1) * HID]
        hbf_ref[hh] = hcol.astype(jnp.bfloat16)
        col = pl.ds(i * BN, BN)
        asrc_ref[pl.ds(hh, 1), col] = jnp.sum(
            hcol * as_ref[hh][None, :], axis=1)[None, :]
        adst_ref[pl.ds(hh, 1), col] = jnp.sum(
            hcol * ad_ref[hh][None, :], axis=1)[None, :]


def _stage_a(x, w1, att_src1, att_dst1):
    return pl.pallas_call(
        _ka_body,
        grid=(NB,),
        in_specs=[
            pl.BlockSpec((BN, D_IN), lambda i: (i, 0)),
            pl.BlockSpec((D_IN, H1 * HID), lambda i: (0, 0)),
            pl.BlockSpec((H1, HID), lambda i: (0, 0)),
            pl.BlockSpec((H1, HID), lambda i: (0, 0)),
        ],
        out_specs=[
            pl.BlockSpec((H1, BN, HID), lambda i: (0, i, 0)),
            pl.BlockSpec((H1, NP), lambda i: (0, 0)),
            pl.BlockSpec((H1, NP), lambda i: (0, 0)),
        ],
        out_shape=[
            jax.ShapeDtypeStruct((H1, NP, HID), jnp.bfloat16),
            jax.ShapeDtypeStruct((H1, NP), jnp.float32),
            jax.ShapeDtypeStruct((H1, NP), jnp.float32),
        ],
    )(x, w1, att_src1, att_dst1)


_MESH = plsc.VectorSubcoreMesh(core_axis_name="c", subcore_axis_name="s")
_SC_PARAMS = pltpu.CompilerParams(needs_layout_passes=False,
                                  use_tc_tiling_on_sc=False)


KE = 2048
NCHE = EP // KE


@functools.partial(
    pl.kernel,
    out_type=(jax.ShapeDtypeStruct((H1 * NS * NP * 4,), jnp.float32),
              jax.ShapeDtypeStruct((H1 * NP,), jnp.float32)),
    mesh=_MESH,
    scratch_types=[
        pltpu.VMEM((2, KE), jnp.int16),
        pltpu.VMEM((2, KE), jnp.int16),
        pltpu.VMEM((N,), jnp.float32),
        pltpu.VMEM((N,), jnp.float32),
        pltpu.VMEM((NP * 2,), jnp.int32),
        pltpu.VMEM((NP * 4,), jnp.float32),
        pltpu.VMEM((NP,), jnp.float32),
        pltpu.SemaphoreType.DMA,
    ],
    compiler_params=_SC_PARAMS,
)
def _kb(hbf_hbm, asrc_hbm, adst_hbm, src_hbm, dst_hbm, acc_hbm, den_hbm,
        src_v, dst_v, as_v, ad_v, tab_v, acc_v, den_v, sem):
    c = lax.axis_index("c")
    s = lax.axis_index("s")
    z16 = jnp.zeros((16,), jnp.float32)
    zi = jnp.zeros((16,), jnp.int32)
    lanes2 = 2 * lax.iota(jnp.int32, 16)

    def idx_fire(ch, b):
        pltpu.async_copy(src_hbm.at[pl.ds(ch * KE, KE)], src_v.at[b], sem)
        pltpu.async_copy(dst_hbm.at[pl.ds(ch * KE, KE)], dst_v.at[b], sem)

    def idx_wait(b):
        pltpu.make_async_copy(src_hbm.at[pl.ds(0, KE)], src_v.at[b],
                              sem).wait()
        pltpu.make_async_copy(dst_hbm.at[pl.ds(0, KE)], dst_v.at[b],
                              sem).wait()

    def head_body(jh, carry):
        h = c * (H1 // NC) + jh
        pltpu.sync_copy(asrc_hbm.at[pl.ds(h * NP, N)], as_v)
        pltpu.sync_copy(adst_hbm.at[pl.ds(h * NP, N)], ad_v)
        pltpu.sync_copy(
            hbf_hbm.at[pl.ds((h * NS + s) * NP * 2, NP * 2)], tab_v)

        def zacc_body(r, carry2):
            acc_v[pl.ds(r * 16, 16)] = z16
            return carry2

        lax.fori_loop(0, NP * 4 // 16, zacc_body, 0)

        def zden_body(r, carry2):
            den_v[pl.ds(r * 16, 16)] = z16
            return carry2

        lax.fori_loop(0, NP // 16, zden_body, 0)

        idx_fire(0, 0)
        idx_fire(1, 1)

        def do_chunk(ch, b):
            idx_wait(b)

            def grp_body(g, carry3):
                sl = pl.ds(g * 32, 32)
                s_e, s_o = plsc.unpack(src_v[b, sl],
                                       format=plsc.PackFormat.INTERLEAVED,
                                       preferred_element_type=jnp.int32)
                d_e, d_o = plsc.unpack(dst_v[b, sl],
                                       format=plsc.PackFormat.INTERLEAVED,
                                       preferred_element_type=jnp.int32)
                for sub, (sidx, didx) in enumerate(((s_e, d_e),
                                                    (s_o, d_o))):
                    a_s = plsc.load_gather(as_v, [sidx])
                    a_d = plsc.load_gather(ad_v, [didx])
                    e = a_s + a_d
                    e = jnp.maximum(e, 0.2 * e)
                    valid = (ch * KE + g * 32 + lanes2 + sub) < E
                    w16 = jnp.where(valid, jnp.exp(e), 0.0)
                    plsc.addupdate_scatter(den_v, [didx], w16)
                    si2 = sidx * 2
                    di4 = didx * 4
                    for pr in range(2):
                        pair = plsc.bitcast(
                            plsc.load_gather(tab_v, [si2 + pr]),
                            jnp.bfloat16)
                        ev, od = plsc.unpack(
                            pair, format=plsc.PackFormat.INTERLEAVED,
                            preferred_element_type=jnp.float32)
                        plsc.addupdate_scatter(acc_v, [di4 + (2 * pr)],
                                               ev * w16)
                        plsc.addupdate_scatter(acc_v, [di4 + (2 * pr + 1)],
                                               od * w16)
                return carry3

            lax.fori_loop(0, KE // 32, grp_body, 0)

        def pair_body(p, carry2):
            ch = 2 * p
            do_chunk(ch, 0)
            idx_fire(ch + 2, 0)
            do_chunk(ch + 1, 1)
            idx_fire(ch + 3, 1)
            return carry2

        lax.fori_loop(0, NCHE // 2 - 1, pair_body, 0)
        do_chunk(NCHE - 2, 0)
        do_chunk(NCHE - 1, 1)
        pltpu.sync_copy(
            acc_v, acc_hbm.at[pl.ds((h * NS + s) * NP * 4, NP * 4)])

        @pl.when(s == 0)
        def _():
            pltpu.sync_copy(den_v, den_hbm.at[pl.ds(h * NP, NP)])

        return carry

    lax.fori_loop(0, H1 // NC, head_body, 0)


def _kc_body(acc_ref, den_ref, b1_ref, w2_ref, as2_ref, ad2_ref,
             h2aug_ref, a2_ref):
    i = pl.program_id(0)
    h2 = jnp.zeros((BN, C2), jnp.float32)
    for hh in range(H1):
        num = acc_ref[hh]
        den = den_ref[hh].reshape(BN, 1)
        t = num / (den + 1e-16) + b1_ref[pl.ds(hh * HID, HID)][None, :]
        t = jnp.where(t > 0, t, jnp.exp(jnp.minimum(t, 0.0)) - 1.0)
        h2 = h2 + jnp.dot(t, w2_ref[pl.ds(hh * HID, HID), :],
                          preferred_element_type=jnp.float32)
    ones = jnp.ones((BN, 1), jnp.float32)
    pad = jnp.zeros((BN, W2AUG - C2 - 1), jnp.float32)
    h2aug_ref[...] = jnp.concatenate([h2, ones, pad], axis=1)
    col = pl.ds(i * BN, BN)
    a2_ref[pl.ds(0, 1), col] = jnp.sum(h2 * as2_ref[0][None, :], axis=1)[None, :]
    a2_ref[pl.ds(1, 1), col] = jnp.sum(h2 * ad2_ref[0][None, :], axis=1)[None, :]


def _stage_c(acc1, den1, b1, w2, att_src2, att_dst2):
    return pl.pallas_call(
        _kc_body,
        grid=(NB,),
        in_specs=[
            pl.BlockSpec((H1, BN, HID), lambda i: (0, i, 0)),
            pl.BlockSpec((H1, BN), lambda i: (0, i)),
            pl.BlockSpec((H1 * HID,), lambda i: (0,)),
            pl.BlockSpec((H1 * HID, C2), lambda i: (0, 0)),
            pl.BlockSpec((1, C2), lambda i: (0, 0)),
            pl.BlockSpec((1, C2), lambda i: (0, 0)),
        ],
        out_specs=[
            pl.BlockSpec((BN, W2AUG), lambda i: (i, 0)),
            pl.BlockSpec((2, NP), lambda i: (0, 0)),
        ],
        out_shape=[
            jax.ShapeDtypeStruct((NP, W2AUG), jnp.float32),
            jax.ShapeDtypeStruct((2, NP), jnp.float32),
        ],
    )(acc1, den1, b1, w2, att_src2, att_dst2)


@functools.partial(
    pl.kernel,
    out_type=jax.ShapeDtypeStruct((NC * NP, W2AUG), jnp.float32),
    mesh=_MESH,
    scratch_types=[
        pltpu.VMEM((E2T,), jnp.int16),
        pltpu.VMEM((E2T,), jnp.int16),
        pltpu.VMEM((N,), jnp.float32),
        pltpu.VMEM((N,), jnp.float32),
        pltpu.VMEM((2, K), jnp.int32),
        pltpu.VMEM((2, K), jnp.int32),
        pltpu.VMEM((2, K), jnp.float32),
        pltpu.VMEM((2, K, W2AUG), jnp.float32),
        pltpu.VMEM((RPT, W2AUG), jnp.float32),
        pltpu.VMEM_SHARED((NP, W2AUG), jnp.float32),
        pltpu.SemaphoreType.DMA,
        pltpu.SemaphoreType.DMA,
    ],
    compiler_params=_SC_PARAMS,
)
def _kd(h2aug_hbm, a2_hbm, src_hbm, dst_hbm, accp_hbm,
        src_v, dst_v, as_v, ad_v, gidx, didx, w_v, rows, zbuf, acc_sp,
        sem, sem2):
    c = lax.axis_index("c")
    s = lax.axis_index("s")
    ebase = (c * NS + s) * E2T
    pltpu.sync_copy(src_hbm.at[pl.ds(ebase, E2T)], src_v)
    pltpu.sync_copy(dst_hbm.at[pl.ds(ebase, E2T)], dst_v)
    pltpu.sync_copy(a2_hbm.at[pl.ds(0, N)], as_v)
    pltpu.sync_copy(a2_hbm.at[pl.ds(NP, N)], ad_v)

    z16 = jnp.zeros((16,), jnp.float32)
    lanes2 = 2 * lax.iota(jnp.int32, 16)

    def zb_body(r, carry):
        zbuf[r, pl.ds(0, 16)] = z16
        return carry

    lax.fori_loop(0, RPT, zb_body, 0)
    rbase = s * RPT
    pltpu.sync_copy(zbuf, acc_sp.at[pl.ds(rbase, RPT)])
    plsc.subcore_barrier()

    def fill_fire(ch, b, wait_sct):
        off = ch * K
        d_regs = []
        for i in range(K // 32):
            sl = pl.ds(off + i * 32, 32)
            s_e, s_o = plsc.unpack(src_v[sl],
                                   format=plsc.PackFormat.INTERLEAVED,
                                   preferred_element_type=jnp.int32)
            d_e, d_o = plsc.unpack(dst_v[sl],
                                   format=plsc.PackFormat.INTERLEAVED,
                                   preferred_element_type=jnp.int32)
            for sub, (sidx, didx16) in enumerate(((s_e, d_e), (s_o, d_o))):
                a_s = plsc.load_gather(as_v, [sidx])
                a_d = plsc.load_gather(ad_v, [didx16])
                e = a_s + a_d
                e = jnp.maximum(e, 0.2 * e)
                valid = (ebase + off + i * 32 + lanes2 + sub) < E
                w_v[b, pl.ds(i * 32 + sub * 16, 16)] = jnp.where(
                    valid, jnp.exp(e), 0.0)
                gidx[b, pl.ds(i * 32 + sub * 16, 16)] = sidx
                d_regs.append((i * 32 + sub * 16, didx16))
        if wait_sct:
            pltpu.make_async_copy(rows.at[b], acc_sp.at[didx.at[b]],
                                  sem2).wait()
        for o, v in d_regs:
            didx[b, pl.ds(o, 16)] = v
        pltpu.async_copy(h2aug_hbm.at[gidx.at[b]], rows.at[b], sem)

    def drain_scatter(b):
        pltpu.make_async_copy(h2aug_hbm.at[gidx.at[b]], rows.at[b],
                              sem).wait()
        zi = jnp.zeros((16,), jnp.int32)

        def mul_body(k4, carry3):
            for dk in range(4):
                k = k4 * 4 + dk
                wk = plsc.load_gather(w_v.at[b], [zi + k])
                rows[b, k, pl.ds(0, 16)] = rows[b, k, pl.ds(0, 16)] * wk
            return carry3

        lax.fori_loop(0, K // 4, mul_body, 0)
        pltpu.async_copy(rows.at[b], acc_sp.at[didx.at[b]], sem2, add=True)

    fill_fire(0, 0, False)
    fill_fire(1, 1, False)

    def pair_body(p, carry2):
        ch = 2 * p
        drain_scatter(0)
        fill_fire(ch + 2, 0, True)
        drain_scatter(1)
        fill_fire(ch + 3, 1, True)
        return carry2

    lax.fori_loop(0, NCH2 // 2 - 1, pair_body, 0)
    drain_scatter(0)
    drain_scatter(1)
    pltpu.make_async_copy(rows.at[0], acc_sp.at[didx.at[0]], sem2).wait()
    pltpu.make_async_copy(rows.at[1], acc_sp.at[didx.at[1]], sem2).wait()
    plsc.subcore_barrier()
    pltpu.sync_copy(acc_sp.at[pl.ds(rbase, RPT)],
                    accp_hbm.at[pl.ds(c * NP + rbase, RPT)])


def _ke_body(accp_ref, b2_ref, out_ref):
    ssum = accp_ref[0] + accp_ref[1]
    out_ref[...] = (ssum[:, 0:C2] / (ssum[:, C2:C2 + 1] + 1e-16)
                    + b2_ref[...][None, :])


def _stage_e(accp, b2):
    return pl.pallas_call(
        _ke_body,
        grid=(NB,),
        in_specs=[
            pl.BlockSpec((2, BN, W2AUG), lambda i: (0, i, 0)),
            pl.BlockSpec((C2,), lambda i: (0,)),
        ],
        out_specs=pl.BlockSpec((BN, C2), lambda i: (i, 0)),
        out_shape=jax.ShapeDtypeStruct((N, C2), jnp.float32),
    )(accp, b2)


def kernel(x, edge_index, W1, att_src1, att_dst1, b1,
           W2, att_src2, att_dst2, b2):
    zpad = jnp.zeros((EP - E,), jnp.int16)
    src = jnp.concatenate([edge_index[0].astype(jnp.int16), zpad])
    dst = jnp.concatenate([edge_index[1].astype(jnp.int16), zpad])
    x_p = jnp.concatenate(
        [x, jnp.zeros((NP - N, D_IN), jnp.float32)], axis=0)
    hbf, asrc, adst = _stage_a(x_p, W1, att_src1, att_dst1)
    hbf4 = hbf.reshape(H1, NP, NS, 4).transpose(0, 2, 1, 3)
    hbf_i32 = jax.lax.bitcast_convert_type(
        hbf4.reshape(H1 * NS * NP * 2, 2), jnp.int32)
    acc1, den1 = _kb(hbf_i32, asrc.reshape(H1 * NP),
                     adst.reshape(H1 * NP), src, dst)
    acc1t = (acc1.reshape(H1, NS, NP, 4).transpose(0, 2, 1, 3)
             .reshape(H1, NP, HID))
    h2aug, a2 = _stage_c(acc1t, den1.reshape(H1, NP), b1, W2,
                         att_src2, att_dst2)
    accp = _kd(h2aug, a2.reshape(2 * NP), src, dst)
    return _stage_e(accp.reshape(NC, NP, W2AUG), b2)


# speedup vs baseline: 3.0807x; 3.0807x over previous
import functools

import jax
import jax.numpy as jnp
from jax import lax
from jax.experimental import pallas as pl
from jax.experimental.pallas import tpu as pltpu
from jax.experimental.pallas import tpu_sc as plsc

N = 10000
E = 320000
D_IN = 128
HID = 64
H1 = 8
C2 = 2
W1AUG = 80
W2AUG = 16

NS = 16
NC = 2
NP = 10240
BN = 1024
NB = NP // BN

K = 128
EP = 327680
E1T = EP // NS
E2T = EP // (NC * NS)
NCH1 = E1T // K
NCH2 = E2T // K
RPT = NP // NS


def _ka_body(x_ref, w1_ref, as_ref, ad_ref, haug_ref, asrc_ref, adst_ref):
    i = pl.program_id(0)
    h = jnp.dot(x_ref[...], w1_ref[...], preferred_element_type=jnp.float32)
    ones = jnp.ones((BN, 1), jnp.float32)
    pad = jnp.zeros((BN, W1AUG - HID - 1), jnp.float32)
    for hh in range(H1):
        hcol = h[:, hh * HID:(hh + 1) * HID]
        haug_ref[hh] = jnp.concatenate([hcol, ones, pad], axis=1)
        col = pl.ds(i * BN, BN)
        asrc_ref[pl.ds(hh, 1), col] = jnp.sum(
            hcol * as_ref[hh][None, :], axis=1)[None, :]
        adst_ref[pl.ds(hh, 1), col] = jnp.sum(
            hcol * ad_ref[hh][None, :], axis=1)[None, :]


def _stage_a(x, w1, att_src1, att_dst1):
    return pl.pallas_call(
        _ka_body,
        grid=(NB,),
        in_specs=[
            pl.BlockSpec((BN, D_IN), lambda i: (i, 0)),
            pl.BlockSpec((D_IN, H1 * HID), lambda i: (0, 0)),
            pl.BlockSpec((H1, HID), lambda i: (0, 0)),
            pl.BlockSpec((H1, HID), lambda i: (0, 0)),
        ],
        out_specs=[
            pl.BlockSpec((H1, BN, W1AUG), lambda i: (0, i, 0)),
            pl.BlockSpec((H1, NP), lambda i: (0, 0)),
            pl.BlockSpec((H1, NP), lambda i: (0, 0)),
        ],
        out_shape=[
            jax.ShapeDtypeStruct((H1, NP, W1AUG), jnp.float32),
            jax.ShapeDtypeStruct((H1, NP), jnp.float32),
            jax.ShapeDtypeStruct((H1, NP), jnp.float32),
        ],
    )(x, w1, att_src1, att_dst1)


_MESH = plsc.VectorSubcoreMesh(core_axis_name="c", subcore_axis_name="s")
_SC_PARAMS = pltpu.CompilerParams(needs_layout_passes=False,
                                  use_tc_tiling_on_sc=False)


@functools.partial(
    pl.kernel,
    out_type=jax.ShapeDtypeStruct((H1 * NP, W1AUG), jnp.float32),
    mesh=_MESH,
    scratch_types=[
        pltpu.VMEM((E1T,), jnp.int16),
        pltpu.VMEM((E1T,), jnp.int16),
        pltpu.VMEM((N,), jnp.float32),
        pltpu.VMEM((N,), jnp.float32),
        pltpu.VMEM((2, K), jnp.int32),
        pltpu.VMEM((2, K), jnp.int32),
        pltpu.VMEM((2, K), jnp.float32),
        pltpu.VMEM((2, K, W1AUG), jnp.float32),
        pltpu.VMEM((64, W1AUG), jnp.float32),
        pltpu.VMEM_SHARED((NP, W1AUG), jnp.float32),
        pltpu.SemaphoreType.DMA,
        pltpu.SemaphoreType.DMA,
    ],
    compiler_params=_SC_PARAMS,
)
def _kb(haug_hbm, asrc_hbm, adst_hbm, src_hbm, dst_hbm, acc_hbm,
        src_v, dst_v, as_v, ad_v, gidx, didx, w_v, rows, zbuf, acc_sp,
        sem, sem2):
    c = lax.axis_index("c")
    s = lax.axis_index("s")
    ebase = s * E1T
    pltpu.sync_copy(src_hbm.at[pl.ds(ebase, E1T)], src_v)
    pltpu.sync_copy(dst_hbm.at[pl.ds(ebase, E1T)], dst_v)

    z16 = jnp.zeros((16,), jnp.float32)
    lanes2 = 2 * lax.iota(jnp.int32, 16)

    def zb_body(r, carry):
        for q in range(W1AUG // 16):
            zbuf[r, pl.ds(q * 16, 16)] = z16
        return carry

    lax.fori_loop(0, 64, zb_body, 0)
    rbase = s * RPT

    def head_body(jh, carry):
        h = c * (H1 // NC) + jh
        hoff = h * NP
        pltpu.sync_copy(asrc_hbm.at[pl.ds(hoff, N)], as_v)
        pltpu.sync_copy(adst_hbm.at[pl.ds(hoff, N)], ad_v)
        for z in range(RPT // 64):
            pltpu.sync_copy(zbuf, acc_sp.at[pl.ds(rbase + z * 64, 64)])
        plsc.subcore_barrier()

        def fill_fire(ch, b, wait_sct):
            off = ch * K
            d_regs = []
            for i in range(K // 32):
                sl = pl.ds(off + i * 32, 32)
                s_e, s_o = plsc.unpack(src_v[sl],
                                       format=plsc.PackFormat.INTERLEAVED,
                                       preferred_element_type=jnp.int32)
                d_e, d_o = plsc.unpack(dst_v[sl],
                                       format=plsc.PackFormat.INTERLEAVED,
                                       preferred_element_type=jnp.int32)
                for sub, (sidx, didx16) in enumerate(((s_e, d_e),
                                                      (s_o, d_o))):
                    a_s = plsc.load_gather(as_v, [sidx])
                    a_d = plsc.load_gather(ad_v, [didx16])
                    e = a_s + a_d
                    e = jnp.maximum(e, 0.2 * e)
                    valid = (ebase + off + i * 32 + lanes2 + sub) < E
                    w_v[b, pl.ds(i * 32 + sub * 16, 16)] = jnp.where(
                        valid, jnp.exp(e), 0.0)
                    gidx[b, pl.ds(i * 32 + sub * 16, 16)] = sidx + hoff
                    d_regs.append((i * 32 + sub * 16, didx16))
            if wait_sct:
                pltpu.make_async_copy(rows.at[b], acc_sp.at[didx.at[b]],
                                      sem2).wait()
            for o, v in d_regs:
                didx[b, pl.ds(o, 16)] = v
            pltpu.async_copy(haug_hbm.at[gidx.at[b]], rows.at[b], sem)

        def drain_scatter(b):
            pltpu.make_async_copy(haug_hbm.at[gidx.at[b]], rows.at[b],
                                  sem).wait()
            zi = jnp.zeros((16,), jnp.int32)

            def mul_body(k4, carry3):
                for dk in range(4):
                    k = k4 * 4 + dk
                    wk = plsc.load_gather(w_v.at[b], [zi + k])
                    for q in range(W1AUG // 16):
                        sl2 = pl.ds(q * 16, 16)
                        rows[b, k, sl2] = rows[b, k, sl2] * wk
                return carry3

            lax.fori_loop(0, K // 4, mul_body, 0)
            pltpu.async_copy(rows.at[b], acc_sp.at[didx.at[b]], sem2,
                             add=True)

        fill_fire(0, 0, False)
        fill_fire(1, 1, False)

        def pair_body(p, carry2):
            ch = 2 * p
            drain_scatter(0)
            fill_fire(ch + 2, 0, True)
            drain_scatter(1)
            fill_fire(ch + 3, 1, True)
            return carry2

        lax.fori_loop(0, NCH1 // 2 - 1, pair_body, 0)
        drain_scatter(0)
        drain_scatter(1)
        pltpu.make_async_copy(rows.at[0], acc_sp.at[didx.at[0]], sem2).wait()
        pltpu.make_async_copy(rows.at[1], acc_sp.at[didx.at[1]], sem2).wait()
        plsc.subcore_barrier()
        pltpu.sync_copy(acc_sp.at[pl.ds(rbase, RPT)],
                        acc_hbm.at[pl.ds(hoff + rbase, RPT)])
        return carry

    lax.fori_loop(0, H1 // NC, head_body, 0)


def _kc_body(acc_ref, b1_ref, w2_ref, as2_ref, ad2_ref, h2aug_ref, a2_ref):
    i = pl.program_id(0)
    h2 = jnp.zeros((BN, C2), jnp.float32)
    for hh in range(H1):
        num = acc_ref[hh, :, 0:HID]
        den = acc_ref[hh, :, HID:HID + 1]
        t = num / (den + 1e-16) + b1_ref[pl.ds(hh * HID, HID)][None, :]
        t = jnp.where(t > 0, t, jnp.exp(jnp.minimum(t, 0.0)) - 1.0)
        h2 = h2 + jnp.dot(t, w2_ref[pl.ds(hh * HID, HID), :],
                          preferred_element_type=jnp.float32)
    ones = jnp.ones((BN, 1), jnp.float32)
    pad = jnp.zeros((BN, W2AUG - C2 - 1), jnp.float32)
    h2aug_ref[...] = jnp.concatenate([h2, ones, pad], axis=1)
    col = pl.ds(i * BN, BN)
    a2_ref[pl.ds(0, 1), col] = jnp.sum(h2 * as2_ref[0][None, :], axis=1)[None, :]
    a2_ref[pl.ds(1, 1), col] = jnp.sum(h2 * ad2_ref[0][None, :], axis=1)[None, :]


def _stage_c(acc1, b1, w2, att_src2, att_dst2):
    return pl.pallas_call(
        _kc_body,
        grid=(NB,),
        in_specs=[
            pl.BlockSpec((H1, BN, W1AUG), lambda i: (0, i, 0)),
            pl.BlockSpec((H1 * HID,), lambda i: (0,)),
            pl.BlockSpec((H1 * HID, C2), lambda i: (0, 0)),
            pl.BlockSpec((1, C2), lambda i: (0, 0)),
            pl.BlockSpec((1, C2), lambda i: (0, 0)),
        ],
        out_specs=[
            pl.BlockSpec((BN, W2AUG), lambda i: (i, 0)),
            pl.BlockSpec((2, NP), lambda i: (0, 0)),
        ],
        out_shape=[
            jax.ShapeDtypeStruct((NP, W2AUG), jnp.float32),
            jax.ShapeDtypeStruct((2, NP), jnp.float32),
        ],
    )(acc1, b1, w2, att_src2, att_dst2)


@functools.partial(
    pl.kernel,
    out_type=jax.ShapeDtypeStruct((NC * NP, W2AUG), jnp.float32),
    mesh=_MESH,
    scratch_types=[
        pltpu.VMEM((E2T,), jnp.int16),
        pltpu.VMEM((E2T,), jnp.int16),
        pltpu.VMEM((N,), jnp.float32),
        pltpu.VMEM((N,), jnp.float32),
        pltpu.VMEM((2, K), jnp.int32),
        pltpu.VMEM((2, K), jnp.int32),
        pltpu.VMEM((2, K), jnp.float32),
        pltpu.VMEM((2, K, W2AUG), jnp.float32),
        pltpu.VMEM((RPT, W2AUG), jnp.float32),
        pltpu.VMEM_SHARED((NP, W2AUG), jnp.float32),
        pltpu.SemaphoreType.DMA,
        pltpu.SemaphoreType.DMA,
    ],
    compiler_params=_SC_PARAMS,
)
def _kd(h2aug_hbm, a2_hbm, src_hbm, dst_hbm, accp_hbm,
        src_v, dst_v, as_v, ad_v, gidx, didx, w_v, rows, zbuf, acc_sp,
        sem, sem2):
    c = lax.axis_index("c")
    s = lax.axis_index("s")
    ebase = (c * NS + s) * E2T
    pltpu.sync_copy(src_hbm.at[pl.ds(ebase, E2T)], src_v)
    pltpu.sync_copy(dst_hbm.at[pl.ds(ebase, E2T)], dst_v)
    pltpu.sync_copy(a2_hbm.at[pl.ds(0, N)], as_v)
    pltpu.sync_copy(a2_hbm.at[pl.ds(NP, N)], ad_v)

    z16 = jnp.zeros((16,), jnp.float32)
    lanes2 = 2 * lax.iota(jnp.int32, 16)

    def zb_body(r, carry):
        zbuf[r, pl.ds(0, 16)] = z16
        return carry

    lax.fori_loop(0, RPT, zb_body, 0)
    rbase = s * RPT
    pltpu.sync_copy(zbuf, acc_sp.at[pl.ds(rbase, RPT)])
    plsc.subcore_barrier()

    def fill_fire(ch, b, wait_sct):
        off = ch * K
        d_regs = []
        for i in range(K // 32):
            sl = pl.ds(off + i * 32, 32)
            s_e, s_o = plsc.unpack(src_v[sl],
                                   format=plsc.PackFormat.INTERLEAVED,
                                   preferred_element_type=jnp.int32)
            d_e, d_o = plsc.unpack(dst_v[sl],
                                   format=plsc.PackFormat.INTERLEAVED,
                                   preferred_element_type=jnp.int32)
            for sub, (sidx, didx16) in enumerate(((s_e, d_e), (s_o, d_o))):
                a_s = plsc.load_gather(as_v, [sidx])
                a_d = plsc.load_gather(ad_v, [didx16])
                e = a_s + a_d
                e = jnp.maximum(e, 0.2 * e)
                valid = (ebase + off + i * 32 + lanes2 + sub) < E
                w_v[b, pl.ds(i * 32 + sub * 16, 16)] = jnp.where(
                    valid, jnp.exp(e), 0.0)
                gidx[b, pl.ds(i * 32 + sub * 16, 16)] = sidx
                d_regs.append((i * 32 + sub * 16, didx16))
        if wait_sct:
            pltpu.make_async_copy(rows.at[b], acc_sp.at[didx.at[b]],
                                  sem2).wait()
        for o, v in d_regs:
            didx[b, pl.ds(o, 16)] = v
        pltpu.async_copy(h2aug_hbm.at[gidx.at[b]], rows.at[b], sem)

    def drain_scatter(b):
        pltpu.make_async_copy(h2aug_hbm.at[gidx.at[b]], rows.at[b],
                              sem).wait()
        zi = jnp.zeros((16,), jnp.int32)

        def mul_body(k4, carry3):
            for dk in range(4):
                k = k4 * 4 + dk
                wk = plsc.load_gather(w_v.at[b], [zi + k])
                rows[b, k, pl.ds(0, 16)] = rows[b, k, pl.ds(0, 16)] * wk
            return carry3

        lax.fori_loop(0, K // 4, mul_body, 0)
        pltpu.async_copy(rows.at[b], acc_sp.at[didx.at[b]], sem2, add=True)

    fill_fire(0, 0, False)
    fill_fire(1, 1, False)

    def pair_body(p, carry2):
        ch = 2 * p
        drain_scatter(0)
        fill_fire(ch + 2, 0, True)
        drain_scatter(1)
        fill_fire(ch + 3, 1, True)
        return carry2

    lax.fori_loop(0, NCH2 // 2 - 1, pair_body, 0)
    drain_scatter(0)
    drain_scatter(1)
    pltpu.make_async_copy(rows.at[0], acc_sp.at[didx.at[0]], sem2).wait()
    pltpu.make_async_copy(rows.at[1], acc_sp.at[didx.at[1]], sem2).wait()
    plsc.subcore_barrier()
    pltpu.sync_copy(acc_sp.at[pl.ds(rbase, RPT)],
                    accp_hbm.at[pl.ds(c * NP + rbase, RPT)])


def _ke_body(accp_ref, b2_ref, out_ref):
    ssum = accp_ref[0] + accp_ref[1]
    out_ref[...] = (ssum[:, 0:C2] / (ssum[:, C2:C2 + 1] + 1e-16)
                    + b2_ref[...][None, :])


def _stage_e(accp, b2):
    return pl.pallas_call(
        _ke_body,
        grid=(NB,),
        in_specs=[
            pl.BlockSpec((2, BN, W2AUG), lambda i: (0, i, 0)),
            pl.BlockSpec((C2,), lambda i: (0,)),
        ],
        out_specs=pl.BlockSpec((BN, C2), lambda i: (i, 0)),
        out_shape=jax.ShapeDtypeStruct((N, C2), jnp.float32),
    )(accp, b2)


def kernel(x, edge_index, W1, att_src1, att_dst1, b1,
           W2, att_src2, att_dst2, b2):
    zpad = jnp.zeros((EP - E,), jnp.int16)
    src = jnp.concatenate([edge_index[0].astype(jnp.int16), zpad])
    dst = jnp.concatenate([edge_index[1].astype(jnp.int16), zpad])
    x_p = jnp.concatenate(
        [x, jnp.zeros((NP - N, D_IN), jnp.float32)], axis=0)
    haug, asrc, adst = _stage_a(x_p, W1, att_src1, att_dst1)
    acc1 = _kb(haug.reshape(H1 * NP, W1AUG), asrc.reshape(H1 * NP),
               adst.reshape(H1 * NP), src, dst)
    h2aug, a2 = _stage_c(acc1.reshape(H1, NP, W1AUG), b1, W2,
                         att_src2, att_dst2)
    accp = _kd(h2aug, a2.reshape(2 * NP), src, dst)
    return _stage_e(accp.reshape(NC, NP, W2AUG), b2)
